# Initial kernel scaffold; baseline (speedup 1.0000x reference)
import functools

import jax
import jax.numpy as jnp
from jax import lax
from jax.experimental import pallas as pl
from jax.experimental.pallas import tpu as pltpu
from jax.experimental.pallas import tpu_sc as plsc

N = 10000
E = 160000
F = 256
D1 = 512
D2 = 256
NSEG = 64

NC = 2
NS = 16
LANES = 16

NPAD = 10240
ROWS_PER_TILE = NPAD // NS
EPAD = 163840
BATCH = 128
NB_W = EPAD // (NC * NS) // BATCH
NB_S = EPAD // NS // BATCH

RB = 1000

_mesh = plsc.VectorSubcoreMesh(
    core_axis_name="c", subcore_axis_name="s", num_cores=NC, num_subcores=NS)

_f32 = jnp.float32
_i32 = jnp.int32


@functools.partial(
    pl.kernel,
    out_type=jax.ShapeDtypeStruct((NC, NPAD, LANES), _f32),
    mesh=_mesh,
    scratch_types=[
        pltpu.VMEM((NB_W, BATCH), _i32),
        pltpu.VMEM((NB_W, BATCH), _f32),
        pltpu.VMEM((BATCH, LANES), _f32),
        pltpu.VMEM((BATCH, LANES), _f32),
        pltpu.VMEM_SHARED((NPAD, LANES), _f32),
    ],
)
def _deg_kernel(dst_hbm, ew_hbm, out_hbm, didx, ewb, rows, zb, acc):
    c = lax.axis_index("c")
    s = lax.axis_index("s")
    wid = s * NC + c
    pltpu.sync_copy(dst_hbm.at[wid], didx)
    pltpu.sync_copy(ew_hbm.at[wid], ewb)

    def _zrow(i, _):
        zb[i, :] = jnp.zeros((LANES,), _f32)
        return 0
    lax.fori_loop(0, BATCH, _zrow, 0)

    base = s * ROWS_PER_TILE
    for k in range(ROWS_PER_TILE // BATCH):
        pltpu.sync_copy(zb, acc.at[pl.ds(base + k * BATCH, BATCH)])
    plsc.subcore_barrier()

    def _batch(b, _):
        def _edge(j, _):
            rows[j, :] = plsc.load_gather(
                ewb, [jnp.full((LANES,), b, _i32), jnp.full((LANES,), j, _i32)])
            return 0
        lax.fori_loop(0, BATCH, _edge, 0)
        pltpu.sync_copy(rows, acc.at[didx.at[b]], add=True)
        return 0
    lax.fori_loop(0, NB_W, _batch, 0)
    plsc.subcore_barrier()

    for k in range(ROWS_PER_TILE // BATCH):
        r0 = base + k * BATCH
        pltpu.sync_copy(acc.at[pl.ds(r0, BATCH)], out_hbm.at[c, pl.ds(r0, BATCH)])


@functools.partial(
    pl.kernel,
    out_type=jax.ShapeDtypeStruct((NC, NPAD, 128), _f32),
    mesh=_mesh,
    scratch_types=[
        pltpu.VMEM((NB_S, BATCH), _i32),
        pltpu.VMEM((NB_S, BATCH), _i32),
        pltpu.VMEM((NB_S, BATCH), _f32),
        pltpu.VMEM((NB_S, BATCH), _i32),
        pltpu.VMEM((NB_S, BATCH), _f32),
        pltpu.VMEM((N,), _f32),
        pltpu.VMEM((BATCH, 128), _f32),
        pltpu.VMEM((BATCH, 128), _f32),
        pltpu.VMEM_SHARED((NPAD, 128), _f32),
        pltpu.SemaphoreType.DMA,
    ],
)
def _spmm_kernel(x_hbm, src_hbm, dst_hbm, ew_hbm, dinv_hbm, out_hbm,
                 sbuf, didx, ewb, gidx, norms, dinvL, rows, zb, acc, sem):
    c = lax.axis_index("c")
    s = lax.axis_index("s")
    pltpu.sync_copy(src_hbm.at[s], sbuf)
    pltpu.sync_copy(dst_hbm.at[s], didx)
    pltpu.sync_copy(ew_hbm.at[s], ewb)
    pltpu.sync_copy(dinv_hbm, dinvL)

    def _pre(b, _):
        for k in range(BATCH // LANES):
            sl = pl.ds(k * LANES, LANES)
            sv = sbuf[b, sl]
            dv = didx[b, sl]
            ev = ewb[b, sl]
            nv = plsc.load_gather(dinvL, [sv]) * ev * plsc.load_gather(dinvL, [dv])
            norms[b, sl] = nv
            gidx[b, sl] = sv * 2 + c
        return 0
    lax.fori_loop(0, NB_S, _pre, 0)

    def _zrow(i, _):
        for k in range(8):
            zb[i, pl.ds(k * LANES, LANES)] = jnp.zeros((LANES,), _f32)
        return 0
    lax.fori_loop(0, BATCH, _zrow, 0)

    base = s * ROWS_PER_TILE
    for k in range(ROWS_PER_TILE // BATCH):
        pltpu.sync_copy(zb, acc.at[pl.ds(base + k * BATCH, BATCH)])
    plsc.subcore_barrier()

    def _batch(b, _):
        pltpu.async_copy(x_hbm.at[gidx.at[b]], rows, sem).wait()

        def _edge(j, _):
            sp = plsc.load_gather(
                norms, [jnp.full((LANES,), b, _i32), jnp.full((LANES,), j, _i32)])
            for k in range(8):
                sl = pl.ds(k * LANES, LANES)
                rows[j, sl] = rows[j, sl] * sp
            return 0
        lax.fori_loop(0, BATCH, _edge, 0)
        pltpu.sync_copy(rows, acc.at[didx.at[b]], add=True)
        return 0
    lax.fori_loop(0, NB_S, _batch, 0)
    plsc.subcore_barrier()

    for k in range(ROWS_PER_TILE // BATCH):
        r0 = base + k * BATCH
        pltpu.sync_copy(acc.at[pl.ds(r0, BATCH)], out_hbm.at[c, pl.ds(r0, BATCH)])


def _dinv_body(d0_ref, d1_ref, dinv_ref, dinv2_ref):
    deg = 1.0 + d0_ref[...][:, 0:1] + d1_ref[...][:, 0:1]
    r = lax.rsqrt(deg)
    dinv_ref[...] = r
    dinv2_ref[...] = r * r


def _dinv_tc(d0, d1):
    return pl.pallas_call(
        _dinv_body,
        out_shape=(jax.ShapeDtypeStruct((N, 1), _f32),
                   jax.ShapeDtypeStruct((N, 1), _f32)),
    )(d0, d1)


def _mm1_body(u1a_ref, u1b_ref, x_ref, dinv2_ref, W1_ref, b1_ref, cw1_ref,
              cb1_ref, W2_ref, z_ref):
    y = jnp.concatenate([u1a_ref[...], u1b_ref[...]], axis=1)
    y = y + dinv2_ref[...] * x_ref[...]
    h = jnp.dot(y, W1_ref[...], preferred_element_type=_f32) + b1_ref[...]
    h = jnp.maximum(h, 0.0)
    z2 = jnp.zeros((RB, 2), _f32)
    hp = jnp.concatenate([z2, h, z2], axis=1)
    acc = jnp.broadcast_to(cb1_ref[:, 0:1], (RB, D1))
    for k in range(5):
        acc = acc + hp[:, k:k + D1] * cw1_ref[:, k:k + 1]
    h1 = jnp.maximum(acc, 0.0)
    z_ref[...] = jnp.dot(h1, W2_ref[...], preferred_element_type=_f32)


def _mm1_tc(u1a, u1b, x, dinv2, W1, b1r, cw1p, cb1p, W2):
    return pl.pallas_call(
        _mm1_body,
        grid=(N // RB,),
        in_specs=[
            pl.BlockSpec((RB, 128), lambda i: (i, 0)),
            pl.BlockSpec((RB, 128), lambda i: (i, 0)),
            pl.BlockSpec((RB, F), lambda i: (i, 0)),
            pl.BlockSpec((RB, 1), lambda i: (i, 0)),
            pl.BlockSpec((F, D1), lambda i: (0, 0)),
            pl.BlockSpec((1, D1), lambda i: (0, 0)),
            pl.BlockSpec((1, 8), lambda i: (0, 0)),
            pl.BlockSpec((1, 8), lambda i: (0, 0)),
            pl.BlockSpec((D1, D2), lambda i: (0, 0)),
        ],
        out_specs=pl.BlockSpec((RB, D2), lambda i: (i, 0)),
        out_shape=jax.ShapeDtypeStruct((N, D2), _f32),
    )(u1a, u1b, x, dinv2, W1, b1r, cw1p, cb1p, W2)


def _mm2_body(u2a_ref, u2b_ref, z_ref, dinv2_ref, b2_ref, cw2_ref, cb2_ref,
              ib_ref, out_ref, sums_acc, cnt_acc):
    i = pl.program_id(0)
    y = jnp.concatenate([u2a_ref[...], u2b_ref[...]], axis=1)
    y = y + dinv2_ref[...] * z_ref[...]
    o = jnp.maximum(y + b2_ref[...], 0.0)
    z1 = jnp.zeros((RB, 1), _f32)
    op = jnp.concatenate([z1, o, z1], axis=1)
    acc = jnp.broadcast_to(cb2_ref[:, 0:1], (RB, D2))
    for k in range(3):
        acc = acc + op[:, k:k + D2] * cw2_ref[:, k:k + 1]
    h2 = jnp.maximum(acc, 0.0)

    onehot = (lax.broadcasted_iota(_i32, (NSEG, RB), 0) == ib_ref[...]).astype(_f32)

    @pl.when(i == 0)
    def _():
        sums_acc[...] = jnp.zeros_like(sums_acc)
        cnt_acc[...] = jnp.zeros_like(cnt_acc)

    sums_acc[...] += jnp.dot(onehot, h2, preferred_element_type=_f32)
    cnt_acc[...] += jnp.sum(onehot, axis=1, keepdims=True)

    @pl.when(i == (N // RB) - 1)
    def _():
        out_ref[...] = sums_acc[...] / jnp.maximum(cnt_acc[:, 0:1], 1.0)


def _mm2_tc(u2a, u2b, z, dinv2, b2r, cw2p, cb2p, ibT):
    return pl.pallas_call(
        _mm2_body,
        grid=(N // RB,),
        in_specs=[
            pl.BlockSpec((RB, 128), lambda i: (i, 0)),
            pl.BlockSpec((RB, 128), lambda i: (i, 0)),
            pl.BlockSpec((RB, D2), lambda i: (i, 0)),
            pl.BlockSpec((RB, 1), lambda i: (i, 0)),
            pl.BlockSpec((1, D2), lambda i: (0, 0)),
            pl.BlockSpec((1, 8), lambda i: (0, 0)),
            pl.BlockSpec((1, 8), lambda i: (0, 0)),
            pl.BlockSpec((1, RB), lambda i: (0, i)),
        ],
        out_specs=pl.BlockSpec((NSEG, D2), lambda i: (0, 0)),
        out_shape=jax.ShapeDtypeStruct((NSEG, D2), _f32),
        scratch_shapes=[pltpu.VMEM((NSEG, D2), _f32),
                        pltpu.VMEM((NSEG, 128), _f32)],
    )(u2a, u2b, z, dinv2, b2r, cw2p, cb2p, ibT)


def kernel(x, edge_index, edge_attr, info_batch, W1, b1, cw1, cb1, W2, b2,
           cw2, cb2):
    src = edge_index[0].astype(_i32)
    dst = edge_index[1].astype(_i32)
    ew = edge_attr.astype(_f32)

    pad = EPAD - E
    srcp = jnp.concatenate([src, jnp.zeros((pad,), _i32)])
    dstp = jnp.concatenate([dst, jnp.zeros((pad,), _i32)])
    ewp = jnp.concatenate([ew, jnp.zeros((pad,), _f32)])

    dst_w = dstp.reshape(NC * NS, NB_W, BATCH)
    ew_w = ewp.reshape(NC * NS, NB_W, BATCH)
    src_s = srcp.reshape(NS, NB_S, BATCH)
    dst_s = dstp.reshape(NS, NB_S, BATCH)
    ew_s = ewp.reshape(NS, NB_S, BATCH)

    degacc = _deg_kernel(dst_w, ew_w)
    dinv, dinv2 = _dinv_tc(degacc[0, :N], degacc[1, :N])
    dinv1d = dinv.reshape(N)

    xflat = x.reshape(2 * N, 128)
    u1 = _spmm_kernel(xflat, src_s, dst_s, ew_s, dinv1d)

    b1r = b1.reshape(1, D1)
    cw1p = jnp.concatenate([cw1, jnp.zeros((3,), _f32)]).reshape(1, 8)
    cb1p = jnp.broadcast_to(cb1.reshape(1, 1), (1, 8))
    z = _mm1_tc(u1[0, :N], u1[1, :N], x, dinv2, W1, b1r, cw1p, cb1p, W2)

    zflat = z.reshape(2 * N, 128)
    u2 = _spmm_kernel(zflat, src_s, dst_s, ew_s, dinv1d)

    b2r = b2.reshape(1, D2)
    cw2p = jnp.concatenate([cw2, jnp.zeros((5,), _f32)]).reshape(1, 8)
    cb2p = jnp.broadcast_to(cb2.reshape(1, 1), (1, 8))
    ibT = info_batch.astype(_i32).reshape(1, N)
    out = _mm2_tc(u2[0, :N], u2[1, :N], z, dinv2, b2r, cw2p, cb2p, ibT)
    return out


# speedup vs baseline: 1.8118x; 1.8118x over previous
import functools

import jax
import jax.numpy as jnp
from jax import lax
from jax.experimental import pallas as pl
from jax.experimental.pallas import tpu as pltpu
from jax.experimental.pallas import tpu_sc as plsc

N = 10000
E = 160000
F = 256
D1 = 512
D2 = 256
NSEG = 64

NC = 2
NS = 16
LANES = 16

PASSES = 3
HALF = 3456
NPAD = PASSES * HALF
ACCR = 3584
TRASH = HALF
EPAD = 163840
BATCH = 128
NB_W = EPAD // (NC * NS) // BATCH
NB_S = EPAD // NS // BATCH

RB = 1000

_f32 = jnp.float32
_i32 = jnp.int32


@functools.lru_cache(maxsize=None)
def _sc_mesh():
    return plsc.VectorSubcoreMesh(
        core_axis_name="c", subcore_axis_name="s",
        num_cores=NC, num_subcores=NS)


def _spmm_body(x_hbm, src_hbm, dst_hbm, ew_hbm, dinv_hbm, out_hbm,
               sbuf, dbuf, ewb, gidx, didxp, norms, dinvL, rows, zb, acc, sem):
    c = lax.axis_index("c")
    s = lax.axis_index("s")
    pltpu.sync_copy(src_hbm.at[s], sbuf)
    pltpu.sync_copy(dst_hbm.at[s], dbuf)
    pltpu.sync_copy(ew_hbm.at[s], ewb)
    pltpu.sync_copy(dinv_hbm, dinvL)

    def _pre(b, _):
        for k in range(BATCH // LANES):
            sl = pl.ds(k * LANES, LANES)
            sv = sbuf[b, sl]
            dv = dbuf[b, sl]
            ev = ewb[b, sl]
            nv = plsc.load_gather(dinvL, [sv]) * ev * plsc.load_gather(dinvL, [dv])
            norms[pl.ds(b * BATCH + k * LANES, LANES)] = nv
            gidx[b, sl] = sv * 2 + c
        return 0
    lax.fori_loop(0, NB_S, _pre, 0)

    def _zrow(i, _):
        for k in range(8):
            zb[i, pl.ds(k * LANES, LANES)] = jnp.zeros((LANES,), _f32)
        return 0
    lax.fori_loop(0, 112, _zrow, 0)

    for p in range(PASSES):
        def _remap(b, _):
            for k in range(BATCH // LANES):
                sl = pl.ds(k * LANES, LANES)
                lv = dbuf[b, sl] - p * HALF
                inh = (lv >= 0) & (lv < HALF)
                didxp[b, sl] = jnp.where(inh, lv, TRASH)
            return 0
        lax.fori_loop(0, NB_S, _remap, 0)

        zbase = s * (ACCR // NS)
        for k in range(2):
            pltpu.sync_copy(zb, acc.at[pl.ds(zbase + k * 112, 112)])
        plsc.subcore_barrier()

        def _batch(b, _):
            pltpu.async_copy(x_hbm.at[gidx.at[b]], rows, sem).wait()

            def _edge(j, _):
                sp = plsc.load_gather(
                    norms, [jnp.full((LANES,), b * BATCH + j, _i32)])
                for k in range(8):
                    sl = pl.ds(k * LANES, LANES)
                    rows[j, sl] = rows[j, sl] * sp
                return 0
            lax.fori_loop(0, BATCH, _edge, 0)
            pltpu.sync_copy(rows, acc.at[didxp.at[b]], add=True)
            return 0
        lax.fori_loop(0, NB_S, _batch, 0)
        plsc.subcore_barrier()

        obase = s * (HALF // NS)
        pltpu.sync_copy(acc.at[pl.ds(obase, HALF // NS)],
                        out_hbm.at[c, pl.ds(p * HALF + obase, HALF // NS)])
        plsc.subcore_barrier()


@functools.lru_cache(maxsize=None)
def _spmm_kernel():
    return pl.kernel(
        _spmm_body,
        out_type=jax.ShapeDtypeStruct((NC, NPAD, 128), _f32),
        mesh=_sc_mesh(),
        compiler_params=pltpu.CompilerParams(needs_layout_passes=False),
        scratch_types=[
            pltpu.VMEM((NB_S, BATCH), _i32),
            pltpu.VMEM((NB_S, BATCH), _i32),
            pltpu.VMEM((NB_S, BATCH), _f32),
            pltpu.VMEM((NB_S, BATCH), _i32),
            pltpu.VMEM((NB_S, BATCH), _i32),
            pltpu.VMEM((NB_S * BATCH,), _f32),
            pltpu.VMEM((N,), _f32),
            pltpu.VMEM((BATCH, 128), _f32),
            pltpu.VMEM((112, 128), _f32),
            pltpu.VMEM_SHARED((ACCR, 128), _f32),
            pltpu.SemaphoreType.DMA,
        ],
    )


def _dinv_body(d0_ref, dinv_ref, dinv2_ref):
    deg = 1.0 + d0_ref[...][:, 0:1]
    r = lax.rsqrt(deg)
    dinv_ref[...] = r
    dinv2_ref[...] = r * r


def _dinv_tc(d0):
    return pl.pallas_call(
        _dinv_body,
        out_shape=(jax.ShapeDtypeStruct((N, 1), _f32),
                   jax.ShapeDtypeStruct((N, 1), _f32)),
    )(d0)


def _mm1_body(flag_ref, ua_ref, ub_ref, x_ref, dinv2_ref, W1_ref, b1_ref,
              cw1_ref, cb1_ref, W2_ref, z_ref):
    @pl.when(flag_ref[0, 0] != 0)
    def _():
        y = jnp.concatenate([ua_ref[...], ub_ref[...]], axis=1)
        y = y + dinv2_ref[...] * x_ref[...]
        h = jnp.dot(y, W1_ref[...], preferred_element_type=_f32) + b1_ref[...]
        h = jnp.maximum(h, 0.0)
        z2 = jnp.zeros((RB, 2), _f32)
        hp = jnp.concatenate([z2, h, z2], axis=1)
        acc = jnp.broadcast_to(cb1_ref[:, 0:1], (RB, D1))
        for k in range(5):
            acc = acc + hp[:, k:k + D1] * cw1_ref[:, k:k + 1]
        h1 = jnp.maximum(acc, 0.0)
        z_ref[...] = jnp.dot(h1, W2_ref[...], preferred_element_type=_f32)


def _mm1_tc(flag, uq, x, dinv2, W1, b1r, cw1p, cb1p, W2):
    return pl.pallas_call(
        _mm1_body,
        grid=(N // RB,),
        in_specs=[
            pl.BlockSpec((1, 1), lambda i: (0, 0)),
            pl.BlockSpec((RB, 128), lambda i: (i, 0)),
            pl.BlockSpec((RB, 128), lambda i: (i, 0)),
            pl.BlockSpec((RB, F), lambda i: (i, 0)),
            pl.BlockSpec((RB, 1), lambda i: (i, 0)),
            pl.BlockSpec((F, D1), lambda i: (0, 0)),
            pl.BlockSpec((1, D1), lambda i: (0, 0)),
            pl.BlockSpec((1, 8), lambda i: (0, 0)),
            pl.BlockSpec((1, 8), lambda i: (0, 0)),
            pl.BlockSpec((D1, D2), lambda i: (0, 0)),
        ],
        out_specs=pl.BlockSpec((RB, D2), lambda i: (i, 0)),
        out_shape=jax.ShapeDtypeStruct((N, D2), _f32),
    )(flag, *uq, x, dinv2, W1, b1r, cw1p, cb1p, W2)


def _mm2_body(flag_ref, ua_ref, ub_ref, z_ref, dinv2_ref, b2_ref,
              cw2_ref, cb2_ref, ib_ref, out_ref, sums_acc, cnt_acc):
    i = pl.program_id(0)

    @pl.when(flag_ref[0, 0] != 0)
    def _():
        y = jnp.concatenate([ua_ref[...], ub_ref[...]], axis=1)
        y = y + dinv2_ref[...] * z_ref[...]
        o = jnp.maximum(y + b2_ref[...], 0.0)
        z1 = jnp.zeros((RB, 1), _f32)
        op = jnp.concatenate([z1, o, z1], axis=1)
        acc = jnp.broadcast_to(cb2_ref[:, 0:1], (RB, D2))
        for k in range(3):
            acc = acc + op[:, k:k + D2] * cw2_ref[:, k:k + 1]
        h2 = jnp.maximum(acc, 0.0)

        onehot = (lax.broadcasted_iota(_i32, (NSEG, RB), 0)
                  == ib_ref[0]).astype(_f32)

        @pl.when(i == 0)
        def _():
            sums_acc[...] = jnp.zeros_like(sums_acc)
            cnt_acc[...] = jnp.zeros_like(cnt_acc)

        sums_acc[...] += jnp.dot(onehot, h2, preferred_element_type=_f32)
        cnt_acc[...] += jnp.sum(onehot, axis=1, keepdims=True)

        @pl.when(i == (N // RB) - 1)
        def _():
            out_ref[...] = sums_acc[...] / jnp.maximum(cnt_acc[:, 0:1], 1.0)


def _mm2_tc(flag, uq, z, dinv2, b2r, cw2p, cb2p, ibT):
    return pl.pallas_call(
        _mm2_body,
        grid=(N // RB,),
        in_specs=[
            pl.BlockSpec((1, 1), lambda i: (0, 0)),
            pl.BlockSpec((RB, 128), lambda i: (i, 0)),
            pl.BlockSpec((RB, 128), lambda i: (i, 0)),
            pl.BlockSpec((RB, D2), lambda i: (i, 0)),
            pl.BlockSpec((RB, 1), lambda i: (i, 0)),
            pl.BlockSpec((1, D2), lambda i: (0, 0)),
            pl.BlockSpec((1, 8), lambda i: (0, 0)),
            pl.BlockSpec((1, 8), lambda i: (0, 0)),
            pl.BlockSpec((1, 1, RB), lambda i: (i, 0, 0)),
        ],
        out_specs=pl.BlockSpec((NSEG, D2), lambda i: (0, 0)),
        out_shape=jax.ShapeDtypeStruct((NSEG, D2), _f32),
        scratch_shapes=[pltpu.VMEM((NSEG, D2), _f32),
                        pltpu.VMEM((NSEG, 128), _f32)],
    )(flag, *uq, z, dinv2, b2r, cw2p, cb2p, ibT)


def kernel(x, edge_index, edge_attr, info_batch, W1, b1, cw1, cb1, W2, b2,
           cw2, cb2):
    src = edge_index[0].astype(_i32)
    dst = edge_index[1].astype(_i32)
    ew = edge_attr.astype(_f32)

    pad = EPAD - E
    srcp = jnp.concatenate([src, jnp.zeros((pad,), _i32)])
    dstp = jnp.concatenate([dst, jnp.zeros((pad,), _i32)])
    ewp = jnp.concatenate([ew, jnp.zeros((pad,), _f32)])

    src_s = srcp.reshape(NS, NB_S, BATCH)
    dst_s = dstp.reshape(NS, NB_S, BATCH)
    ew_s = ewp.reshape(NS, NB_S, BATCH)

    b1r = b1.reshape(1, D1)
    cw1p = jnp.concatenate([cw1, jnp.zeros((3,), _f32)]).reshape(1, 8)
    cb1p = jnp.broadcast_to(cb1.reshape(1, 1), (1, 8))
    b2r = b2.reshape(1, D2)
    cw2p = jnp.concatenate([cw2, jnp.zeros((5,), _f32)]).reshape(1, 8)
    cb2p = jnp.broadcast_to(cb2.reshape(1, 1), (1, 8))
    ibT = info_batch.astype(_i32).reshape(N // RB, 1, RB)

    spmm = _spmm_kernel()
    ones2n = jnp.ones((2 * N, 128), _f32)
    ones_n = jnp.ones((N,), _f32)

    steps = 2 + (ew[0] < 0.5).astype(_i32) + (ew[0] >= 0.5).astype(_i32)

    def _step(i, carry):
        h, dinv, dinv2, pooled, flags = carry
        fdeg = flags[0]
        f1 = flags[1].reshape(1, 1)
        f2 = flags[2].reshape(1, 1)
        xin = jnp.where(fdeg == 1, ones2n, h.reshape(2 * N, 128))
        dinvin = jnp.where(fdeg == 1, ones_n, dinv.reshape(N))
        u = spmm(xin, src_s, dst_s, ew_s, dinvin)
        uq = (u[0, :N], u[1, :N])
        dinv_new, dinv2_new = _dinv_tc(u[0, :N])
        dinv = jnp.where(fdeg == 1, dinv_new, dinv)
        dinv2 = jnp.where(fdeg == 1, dinv2_new, dinv2)
        z = _mm1_tc(f1, uq, h, dinv2, W1, b1r, cw1p, cb1p, W2)
        pooled_new = _mm2_tc(f2, uq, h, dinv2, b2r, cw2p, cb2p, ibT)
        h = jnp.where(f1[0, 0] == 1, z, h)
        pooled = jnp.where(f2[0, 0] == 1, pooled_new, pooled)
        return (h, dinv, dinv2, pooled, jnp.roll(flags, 1))

    carry0 = (x, jnp.zeros((N, 1), _f32), jnp.zeros((N, 1), _f32),
              jnp.zeros((NSEG, D2), _f32),
              jnp.array([1, 0, 0], _i32))
    _, _, _, out, _ = lax.fori_loop(0, steps, _step, carry0)
    return out


# speedup vs baseline: 1.9965x; 1.1019x over previous
import functools

import jax
import jax.numpy as jnp
from jax import lax
from jax.experimental import pallas as pl
from jax.experimental.pallas import tpu as pltpu
from jax.experimental.pallas import tpu_sc as plsc

N = 10000
E = 160000
F = 256
D1 = 512
D2 = 256
NSEG = 64

NC = 2
NS = 16
LANES = 16

PASSES = 3
HALF = 3456
NPAD = PASSES * HALF
ACCR = 3584
TRASH = HALF
EPAD = 163840
BATCH = 128
NB_W = EPAD // (NC * NS) // BATCH
NB_S = EPAD // NS // BATCH

RB = 1000

_f32 = jnp.float32
_i32 = jnp.int32


@functools.lru_cache(maxsize=None)
def _sc_mesh():
    return plsc.VectorSubcoreMesh(
        core_axis_name="c", subcore_axis_name="s",
        num_cores=NC, num_subcores=NS)


def _spmm_body(x_hbm, src_hbm, dst_hbm, ew_hbm, dinv_hbm, out_hbm,
               sbuf, dbuf, ewb, gidx, didxp, norms, dinvL, rows, zb, acc, sem):
    c = lax.axis_index("c")
    s = lax.axis_index("s")
    pltpu.sync_copy(src_hbm.at[s], sbuf)
    pltpu.sync_copy(dst_hbm.at[s], dbuf)
    pltpu.sync_copy(ew_hbm.at[s], ewb)
    pltpu.sync_copy(dinv_hbm, dinvL)

    def _pre(b, _):
        for k in range(BATCH // LANES):
            sl = pl.ds(k * LANES, LANES)
            sv = sbuf[b, sl]
            dv = dbuf[b, sl]
            ev = ewb[b, sl]
            nv = plsc.load_gather(dinvL, [sv]) * ev * plsc.load_gather(dinvL, [dv])
            norms[pl.ds(b * BATCH + k * LANES, LANES)] = nv
            gidx[b, sl] = sv * 2 + c
        return 0
    lax.fori_loop(0, NB_S, _pre, 0)

    def _zrow(i, _):
        for k in range(8):
            zb[i, pl.ds(k * LANES, LANES)] = jnp.zeros((LANES,), _f32)
        return 0
    lax.fori_loop(0, 112, _zrow, 0)

    for p in range(PASSES):
        def _remap(b, _):
            for k in range(BATCH // LANES):
                sl = pl.ds(k * LANES, LANES)
                lv = dbuf[b, sl] - p * HALF
                inh = (lv >= 0) & (lv < HALF)
                didxp[b, sl] = jnp.where(inh, lv, TRASH)
            return 0
        lax.fori_loop(0, NB_S, _remap, 0)

        zbase = s * (ACCR // NS)
        for k in range(2):
            pltpu.sync_copy(zb, acc.at[pl.ds(zbase + k * 112, 112)])
        plsc.subcore_barrier()

        def _batch(b, _):
            pltpu.async_copy(x_hbm.at[gidx.at[b]], rows, sem).wait()

            def _edge4(jj, _):
                j0 = jj * 4
                sps = [plsc.load_gather(
                    norms, [jnp.full((LANES,), b * BATCH + j0 + e, _i32)])
                    for e in range(4)]
                for e in range(4):
                    for k in range(8):
                        sl = pl.ds(k * LANES, LANES)
                        rows[j0 + e, sl] = rows[j0 + e, sl] * sps[e]
                return 0
            lax.fori_loop(0, BATCH // 4, _edge4, 0)
            pltpu.sync_copy(rows, acc.at[didxp.at[b]], add=True)
            return 0
        lax.fori_loop(0, NB_S, _batch, 0)
        plsc.subcore_barrier()

        obase = s * (HALF // NS)
        pltpu.sync_copy(acc.at[pl.ds(obase, HALF // NS)],
                        out_hbm.at[c, pl.ds(p * HALF + obase, HALF // NS)])
        plsc.subcore_barrier()


@functools.lru_cache(maxsize=None)
def _spmm_kernel():
    return pl.kernel(
        _spmm_body,
        out_type=jax.ShapeDtypeStruct((NC, NPAD, 128), _f32),
        mesh=_sc_mesh(),
        compiler_params=pltpu.CompilerParams(needs_layout_passes=False),
        scratch_types=[
            pltpu.VMEM((NB_S, BATCH), _i32),
            pltpu.VMEM((NB_S, BATCH), _i32),
            pltpu.VMEM((NB_S, BATCH), _f32),
            pltpu.VMEM((NB_S, BATCH), _i32),
            pltpu.VMEM((NB_S, BATCH), _i32),
            pltpu.VMEM((NB_S * BATCH,), _f32),
            pltpu.VMEM((N,), _f32),
            pltpu.VMEM((BATCH, 128), _f32),
            pltpu.VMEM((112, 128), _f32),
            pltpu.VMEM_SHARED((ACCR, 128), _f32),
            pltpu.SemaphoreType.DMA,
        ],
    )


def _dinv_body(d0_ref, dinv_ref, dinv2_ref):
    deg = 1.0 + d0_ref[...][:, 0:1]
    r = lax.rsqrt(deg)
    dinv_ref[...] = r
    dinv2_ref[...] = r * r


def _dinv_tc(d0):
    return pl.pallas_call(
        _dinv_body,
        out_shape=(jax.ShapeDtypeStruct((N, 1), _f32),
                   jax.ShapeDtypeStruct((N, 1), _f32)),
    )(d0)


def _mm1_body(flag_ref, ua_ref, ub_ref, x_ref, dinv2_ref, W1_ref, b1_ref,
              cw1_ref, cb1_ref, W2_ref, z_ref):
    @pl.when(flag_ref[0, 0] != 0)
    def _():
        y = jnp.concatenate([ua_ref[...], ub_ref[...]], axis=1)
        y = y + dinv2_ref[...] * x_ref[...]
        h = jnp.dot(y, W1_ref[...], preferred_element_type=_f32) + b1_ref[...]
        h = jnp.maximum(h, 0.0)
        z2 = jnp.zeros((RB, 2), _f32)
        hp = jnp.concatenate([z2, h, z2], axis=1)
        acc = jnp.broadcast_to(cb1_ref[:, 0:1], (RB, D1))
        for k in range(5):
            acc = acc + hp[:, k:k + D1] * cw1_ref[:, k:k + 1]
        h1 = jnp.maximum(acc, 0.0)
        z_ref[...] = jnp.dot(h1, W2_ref[...], preferred_element_type=_f32)


def _mm1_tc(flag, uq, x, dinv2, W1, b1r, cw1p, cb1p, W2):
    return pl.pallas_call(
        _mm1_body,
        grid=(N // RB,),
        in_specs=[
            pl.BlockSpec((1, 1), lambda i: (0, 0)),
            pl.BlockSpec((RB, 128), lambda i: (i, 0)),
            pl.BlockSpec((RB, 128), lambda i: (i, 0)),
            pl.BlockSpec((RB, F), lambda i: (i, 0)),
            pl.BlockSpec((RB, 1), lambda i: (i, 0)),
            pl.BlockSpec((F, D1), lambda i: (0, 0)),
            pl.BlockSpec((1, D1), lambda i: (0, 0)),
            pl.BlockSpec((1, 8), lambda i: (0, 0)),
            pl.BlockSpec((1, 8), lambda i: (0, 0)),
            pl.BlockSpec((D1, D2), lambda i: (0, 0)),
        ],
        out_specs=pl.BlockSpec((RB, D2), lambda i: (i, 0)),
        out_shape=jax.ShapeDtypeStruct((N, D2), _f32),
    )(flag, *uq, x, dinv2, W1, b1r, cw1p, cb1p, W2)


def _mm2_body(flag_ref, ua_ref, ub_ref, z_ref, dinv2_ref, b2_ref,
              cw2_ref, cb2_ref, ib_ref, out_ref, sums_acc, cnt_acc):
    i = pl.program_id(0)

    @pl.when(flag_ref[0, 0] != 0)
    def _():
        y = jnp.concatenate([ua_ref[...], ub_ref[...]], axis=1)
        y = y + dinv2_ref[...] * z_ref[...]
        o = jnp.maximum(y + b2_ref[...], 0.0)
        z1 = jnp.zeros((RB, 1), _f32)
        op = jnp.concatenate([z1, o, z1], axis=1)
        acc = jnp.broadcast_to(cb2_ref[:, 0:1], (RB, D2))
        for k in range(3):
            acc = acc + op[:, k:k + D2] * cw2_ref[:, k:k + 1]
        h2 = jnp.maximum(acc, 0.0)

        onehot = (lax.broadcasted_iota(_i32, (NSEG, RB), 0)
                  == ib_ref[0]).astype(_f32)

        @pl.when(i == 0)
        def _():
            sums_acc[...] = jnp.zeros_like(sums_acc)
            cnt_acc[...] = jnp.zeros_like(cnt_acc)

        sums_acc[...] += jnp.dot(onehot, h2, preferred_element_type=_f32)
        cnt_acc[...] += jnp.sum(onehot, axis=1, keepdims=True)

        @pl.when(i == (N // RB) - 1)
        def _():
            out_ref[...] = sums_acc[...] / jnp.maximum(cnt_acc[:, 0:1], 1.0)


def _mm2_tc(flag, uq, z, dinv2, b2r, cw2p, cb2p, ibT):
    return pl.pallas_call(
        _mm2_body,
        grid=(N // RB,),
        in_specs=[
            pl.BlockSpec((1, 1), lambda i: (0, 0)),
            pl.BlockSpec((RB, 128), lambda i: (i, 0)),
            pl.BlockSpec((RB, 128), lambda i: (i, 0)),
            pl.BlockSpec((RB, D2), lambda i: (i, 0)),
            pl.BlockSpec((RB, 1), lambda i: (i, 0)),
            pl.BlockSpec((1, D2), lambda i: (0, 0)),
            pl.BlockSpec((1, 8), lambda i: (0, 0)),
            pl.BlockSpec((1, 8), lambda i: (0, 0)),
            pl.BlockSpec((1, 1, RB), lambda i: (i, 0, 0)),
        ],
        out_specs=pl.BlockSpec((NSEG, D2), lambda i: (0, 0)),
        out_shape=jax.ShapeDtypeStruct((NSEG, D2), _f32),
        scratch_shapes=[pltpu.VMEM((NSEG, D2), _f32),
                        pltpu.VMEM((NSEG, 128), _f32)],
    )(flag, *uq, z, dinv2, b2r, cw2p, cb2p, ibT)


def kernel(x, edge_index, edge_attr, info_batch, W1, b1, cw1, cb1, W2, b2,
           cw2, cb2):
    src = edge_index[0].astype(_i32)
    dst = edge_index[1].astype(_i32)
    ew = edge_attr.astype(_f32)

    pad = EPAD - E
    srcp = jnp.concatenate([src, jnp.zeros((pad,), _i32)])
    dstp = jnp.concatenate([dst, jnp.zeros((pad,), _i32)])
    ewp = jnp.concatenate([ew, jnp.zeros((pad,), _f32)])

    src_s = srcp.reshape(NS, NB_S, BATCH)
    dst_s = dstp.reshape(NS, NB_S, BATCH)
    ew_s = ewp.reshape(NS, NB_S, BATCH)

    b1r = b1.reshape(1, D1)
    cw1p = jnp.concatenate([cw1, jnp.zeros((3,), _f32)]).reshape(1, 8)
    cb1p = jnp.broadcast_to(cb1.reshape(1, 1), (1, 8))
    b2r = b2.reshape(1, D2)
    cw2p = jnp.concatenate([cw2, jnp.zeros((5,), _f32)]).reshape(1, 8)
    cb2p = jnp.broadcast_to(cb2.reshape(1, 1), (1, 8))
    ibT = info_batch.astype(_i32).reshape(N // RB, 1, RB)

    spmm = _spmm_kernel()
    ones2n = jnp.ones((2 * N, 128), _f32)
    ones_n = jnp.ones((N,), _f32)

    steps = 2 + (ew[0] < 0.5).astype(_i32) + (ew[0] >= 0.5).astype(_i32)

    def _step(i, carry):
        h, dinv, dinv2, pooled, flags = carry
        fdeg = flags[0]
        f1 = flags[1].reshape(1, 1)
        f2 = flags[2].reshape(1, 1)
        xin = jnp.where(fdeg == 1, ones2n, h.reshape(2 * N, 128))
        dinvin = jnp.where(fdeg == 1, ones_n, dinv.reshape(N))
        u = spmm(xin, src_s, dst_s, ew_s, dinvin)
        uq = (u[0, :N], u[1, :N])
        dinv_new, dinv2_new = _dinv_tc(u[0, :N])
        dinv = jnp.where(fdeg == 1, dinv_new, dinv)
        dinv2 = jnp.where(fdeg == 1, dinv2_new, dinv2)
        z = _mm1_tc(f1, uq, h, dinv2, W1, b1r, cw1p, cb1p, W2)
        pooled_new = _mm2_tc(f2, uq, h, dinv2, b2r, cw2p, cb2p, ibT)
        h = jnp.where(f1[0, 0] == 1, z, h)
        pooled = jnp.where(f2[0, 0] == 1, pooled_new, pooled)
        return (h, dinv, dinv2, pooled, jnp.roll(flags, 1))

    carry0 = (x, jnp.zeros((N, 1), _f32), jnp.zeros((N, 1), _f32),
              jnp.zeros((NSEG, D2), _f32),
              jnp.array([1, 0, 0], _i32))
    _, _, _, out, _ = lax.fori_loop(0, steps, _step, carry0)
    return out


# speedup vs baseline: 2.6752x; 1.3400x over previous
import functools

import jax
import jax.numpy as jnp
from jax import lax
from jax.experimental import pallas as pl
from jax.experimental.pallas import tpu as pltpu
from jax.experimental.pallas import tpu_sc as plsc

N = 10000
E = 160000
F = 256
D1 = 512
D2 = 256
NSEG = 64

NC = 2
NS = 16
LANES = 16

PASSES = 5
HALF = 2048
NPAD = PASSES * HALF
ACCR = HALF
EPAD = 163840
BATCH = 128
NB_S = EPAD // NS // BATCH
CH = 24

RB = 1000

_f32 = jnp.float32
_i32 = jnp.int32


@functools.lru_cache(maxsize=None)
def _sc_mesh():
    return plsc.VectorSubcoreMesh(
        core_axis_name="c", subcore_axis_name="s",
        num_cores=NC, num_subcores=NS)


def _spmm_body(x_hbm, src_hbm, dst_hbm, ew_hbm, dinv_hbm, out_hbm,
               sbuf, dbuf, ewb, gidx, didxp, gidxp, norms, normsp, gidxf,
               didxf, dinvL, rows, acc, semga):
    c = lax.axis_index("c")
    s = lax.axis_index("s")
    pltpu.sync_copy(src_hbm.at[s], sbuf)
    pltpu.sync_copy(dst_hbm.at[s], dbuf)
    pltpu.sync_copy(ew_hbm.at[s], ewb)
    pltpu.sync_copy(dinv_hbm, dinvL)

    def _pre(b, _):
        for k in range(BATCH // LANES):
            sl = pl.ds(k * LANES, LANES)
            sv = sbuf[b, sl]
            dv = dbuf[b, sl]
            ev = ewb[b, sl]
            nv = plsc.load_gather(dinvL, [sv]) * ev * plsc.load_gather(dinvL, [dv])
            norms[pl.ds(b * BATCH + k * LANES, LANES)] = nv
            gidx[b, sl] = sv * 2 + c
        return 0
    lax.fori_loop(0, NB_S, _pre, 0)


    def _pass(p, _):
        def _fill(i, _):
            fl = pl.ds(i * LANES, LANES)
            gidxf[fl] = jnp.zeros((LANES,), _i32)
            normsp[fl] = jnp.zeros((LANES,), _f32)
            didxf[fl] = jnp.zeros((LANES,), _i32)
            return 0
        lax.fori_loop(0, NB_S * BATCH // LANES, _fill, 0)

        def _compact(g, cnt):
            b = g // (BATCH // LANES)
            k = g - b * (BATCH // LANES)
            sl = pl.ds(k * LANES, LANES)
            fl = pl.ds(g * LANES, LANES)
            lv = dbuf[b, sl] - p * HALF
            inh = (lv >= 0) & (lv < HALF)
            plsc.store_compressed(didxf.at[pl.ds(cnt, LANES)], lv, mask=inh)
            plsc.store_compressed(gidxf.at[pl.ds(cnt, LANES)],
                                  gidx[b, sl], mask=inh)
            plsc.store_compressed(normsp.at[pl.ds(cnt, LANES)],
                                  norms[fl], mask=inh)
            npop = plsc.all_reduce_population_count(inh)
            return cnt + jnp.max(npop)
        cnt = lax.fori_loop(0, NB_S * BATCH // LANES, _compact, 0)
        nb_p = (cnt + BATCH - 1) // BATCH

        def _zrow(i, _):
            for k in range(8):
                rows[i, pl.ds(k * LANES, LANES)] = jnp.zeros((LANES,), _f32)
            return 0
        lax.fori_loop(0, 32, _zrow, 0)
        zbase = s * (ACCR // NS)

        def _zcp(i, _):
            pltpu.sync_copy(rows.at[pl.ds(0, 32)],
                            acc.at[pl.ds(zbase + i * 32, 32)])
            return 0
        lax.fori_loop(0, ACCR // NS // 32, _zcp, 0)
        plsc.subcore_barrier()

        nq = (nb_p + CH - 1) // CH

        def _chunk(q, _):
            cbase = q * CH
            ncb = jnp.minimum(nb_p - cbase, CH)

            def _relay(bb, _):
                b = cbase + bb
                for k in range(BATCH // LANES):
                    sl = pl.ds(k * LANES, LANES)
                    fl = pl.ds(b * BATCH + k * LANES, LANES)
                    didxp[bb, sl] = didxf[fl]
                    gidxp[bb, sl] = gidxf[fl]
                return 0
            lax.fori_loop(0, ncb, _relay, 0)

            def _batch(bb, _):
                pltpu.async_copy(x_hbm.at[gidxp.at[bb]], rows, semga).wait()
                ebase = (cbase + bb) * BATCH

                def _edge4(jj, _):
                    j0 = jj * 4
                    sps = [plsc.load_gather(
                        normsp, [jnp.full((LANES,), ebase + j0 + e, _i32)])
                        for e in range(4)]
                    for e in range(4):
                        for k in range(8):
                            sl = pl.ds(k * LANES, LANES)
                            rows[j0 + e, sl] = rows[j0 + e, sl] * sps[e]
                    return 0
                lax.fori_loop(0, BATCH // 4, _edge4, 0)
                pltpu.sync_copy(rows, acc.at[didxp.at[bb]], add=True)
                return 0
            lax.fori_loop(0, ncb, _batch, 0)
            return 0
        lax.fori_loop(0, nq, _chunk, 0)
        plsc.subcore_barrier()

        obase = s * (HALF // NS)

        def _ocp(i, _):
            pltpu.sync_copy(
                acc.at[pl.ds(obase + i * 32, 32)],
                out_hbm.at[c, pl.ds(p * HALF + obase + i * 32, 32)])
            return 0
        lax.fori_loop(0, HALF // NS // 32, _ocp, 0)
        plsc.subcore_barrier()
        return 0

    lax.fori_loop(0, PASSES, _pass, 0)


@functools.lru_cache(maxsize=None)
def _spmm_kernel():
    return pl.kernel(
        _spmm_body,
        out_type=jax.ShapeDtypeStruct((NC, NPAD, 128), _f32),
        mesh=_sc_mesh(),
        compiler_params=pltpu.CompilerParams(needs_layout_passes=False),
        scratch_types=[
            pltpu.VMEM((NB_S, BATCH), _i32),
            pltpu.VMEM((NB_S, BATCH), _i32),
            pltpu.VMEM((NB_S, BATCH), _f32),
            pltpu.VMEM((NB_S, BATCH), _i32),
            pltpu.VMEM((CH, BATCH), _i32),
            pltpu.VMEM((CH, BATCH), _i32),
            pltpu.VMEM((NB_S * BATCH,), _f32),
            pltpu.VMEM((NB_S * BATCH,), _f32),
            pltpu.VMEM((NB_S * BATCH,), _i32),
            pltpu.VMEM((NB_S * BATCH,), _i32),
            pltpu.VMEM((N,), _f32),
            pltpu.VMEM((BATCH, 128), _f32),
            pltpu.VMEM_SHARED((ACCR, 128), _f32),
            pltpu.SemaphoreType.DMA,
        ],
    )


def _dinv_body(d0_ref, dinv_ref, dinv2_ref):
    deg = 1.0 + d0_ref[...][:, 0:1]
    r = lax.rsqrt(deg)
    dinv_ref[...] = r
    dinv2_ref[...] = r * r


def _dinv_tc(d0):
    return pl.pallas_call(
        _dinv_body,
        out_shape=(jax.ShapeDtypeStruct((N, 1), _f32),
                   jax.ShapeDtypeStruct((N, 1), _f32)),
    )(d0)


def _mm1_body(flag_ref, ua_ref, ub_ref, x_ref, dinv2_ref, W1_ref, b1_ref,
              cw1_ref, cb1_ref, W2_ref, z_ref):
    @pl.when(flag_ref[0, 0] != 0)
    def _():
        y = jnp.concatenate([ua_ref[...], ub_ref[...]], axis=1)
        y = y + dinv2_ref[...] * x_ref[...]
        h = jnp.dot(y, W1_ref[...], preferred_element_type=_f32) + b1_ref[...]
        h = jnp.maximum(h, 0.0)
        z2 = jnp.zeros((RB, 2), _f32)
        hp = jnp.concatenate([z2, h, z2], axis=1)
        acc = jnp.broadcast_to(cb1_ref[:, 0:1], (RB, D1))
        for k in range(5):
            acc = acc + hp[:, k:k + D1] * cw1_ref[:, k:k + 1]
        h1 = jnp.maximum(acc, 0.0)
        z_ref[...] = jnp.dot(h1, W2_ref[...], preferred_element_type=_f32)


def _mm1_tc(flag, uq, x, dinv2, W1, b1r, cw1p, cb1p, W2):
    return pl.pallas_call(
        _mm1_body,
        grid=(N // RB,),
        in_specs=[
            pl.BlockSpec((1, 1), lambda i: (0, 0)),
            pl.BlockSpec((RB, 128), lambda i: (i, 0)),
            pl.BlockSpec((RB, 128), lambda i: (i, 0)),
            pl.BlockSpec((RB, F), lambda i: (i, 0)),
            pl.BlockSpec((RB, 1), lambda i: (i, 0)),
            pl.BlockSpec((F, D1), lambda i: (0, 0)),
            pl.BlockSpec((1, D1), lambda i: (0, 0)),
            pl.BlockSpec((1, 8), lambda i: (0, 0)),
            pl.BlockSpec((1, 8), lambda i: (0, 0)),
            pl.BlockSpec((D1, D2), lambda i: (0, 0)),
        ],
        out_specs=pl.BlockSpec((RB, D2), lambda i: (i, 0)),
        out_shape=jax.ShapeDtypeStruct((N, D2), _f32),
    )(flag, *uq, x, dinv2, W1, b1r, cw1p, cb1p, W2)


def _mm2_body(flag_ref, ua_ref, ub_ref, z_ref, dinv2_ref, b2_ref,
              cw2_ref, cb2_ref, ib_ref, out_ref, sums_acc, cnt_acc):
    i = pl.program_id(0)

    @pl.when(flag_ref[0, 0] != 0)
    def _():
        y = jnp.concatenate([ua_ref[...], ub_ref[...]], axis=1)
        y = y + dinv2_ref[...] * z_ref[...]
        o = jnp.maximum(y + b2_ref[...], 0.0)
        z1 = jnp.zeros((RB, 1), _f32)
        op = jnp.concatenate([z1, o, z1], axis=1)
        acc = jnp.broadcast_to(cb2_ref[:, 0:1], (RB, D2))
        for k in range(3):
            acc = acc + op[:, k:k + D2] * cw2_ref[:, k:k + 1]
        h2 = jnp.maximum(acc, 0.0)

        onehot = (lax.broadcasted_iota(_i32, (NSEG, RB), 0)
                  == ib_ref[0]).astype(_f32)

        @pl.when(i == 0)
        def _():
            sums_acc[...] = jnp.zeros_like(sums_acc)
            cnt_acc[...] = jnp.zeros_like(cnt_acc)

        sums_acc[...] += jnp.dot(onehot, h2, preferred_element_type=_f32)
        cnt_acc[...] += jnp.sum(onehot, axis=1, keepdims=True)

        @pl.when(i == (N // RB) - 1)
        def _():
            out_ref[...] = sums_acc[...] / jnp.maximum(cnt_acc[:, 0:1], 1.0)


def _mm2_tc(flag, uq, z, dinv2, b2r, cw2p, cb2p, ibT):
    return pl.pallas_call(
        _mm2_body,
        grid=(N // RB,),
        in_specs=[
            pl.BlockSpec((1, 1), lambda i: (0, 0)),
            pl.BlockSpec((RB, 128), lambda i: (i, 0)),
            pl.BlockSpec((RB, 128), lambda i: (i, 0)),
            pl.BlockSpec((RB, D2), lambda i: (i, 0)),
            pl.BlockSpec((RB, 1), lambda i: (i, 0)),
            pl.BlockSpec((1, D2), lambda i: (0, 0)),
            pl.BlockSpec((1, 8), lambda i: (0, 0)),
            pl.BlockSpec((1, 8), lambda i: (0, 0)),
            pl.BlockSpec((1, 1, RB), lambda i: (i, 0, 0)),
        ],
        out_specs=pl.BlockSpec((NSEG, D2), lambda i: (0, 0)),
        out_shape=jax.ShapeDtypeStruct((NSEG, D2), _f32),
        scratch_shapes=[pltpu.VMEM((NSEG, D2), _f32),
                        pltpu.VMEM((NSEG, 128), _f32)],
    )(flag, *uq, z, dinv2, b2r, cw2p, cb2p, ibT)


def kernel(x, edge_index, edge_attr, info_batch, W1, b1, cw1, cb1, W2, b2,
           cw2, cb2):
    src = edge_index[0].astype(_i32)
    dst = edge_index[1].astype(_i32)
    ew = edge_attr.astype(_f32)

    pad = EPAD - E
    srcp = jnp.concatenate([src, jnp.zeros((pad,), _i32)])
    dstp = jnp.concatenate([dst, jnp.zeros((pad,), _i32)])
    ewp = jnp.concatenate([ew, jnp.zeros((pad,), _f32)])

    src_s = srcp.reshape(NS, NB_S, BATCH)
    dst_s = dstp.reshape(NS, NB_S, BATCH)
    ew_s = ewp.reshape(NS, NB_S, BATCH)

    b1r = b1.reshape(1, D1)
    cw1p = jnp.concatenate([cw1, jnp.zeros((3,), _f32)]).reshape(1, 8)
    cb1p = jnp.broadcast_to(cb1.reshape(1, 1), (1, 8))
    b2r = b2.reshape(1, D2)
    cw2p = jnp.concatenate([cw2, jnp.zeros((5,), _f32)]).reshape(1, 8)
    cb2p = jnp.broadcast_to(cb2.reshape(1, 1), (1, 8))
    ibT = info_batch.astype(_i32).reshape(N // RB, 1, RB)

    spmm = _spmm_kernel()
    ones2n = jnp.ones((2 * N, 128), _f32)
    ones_n = jnp.ones((N,), _f32)

    steps = 2 + (ew[0] < 0.5).astype(_i32) + (ew[0] >= 0.5).astype(_i32)

    def _step(i, carry):
        h, dinv, dinv2, pooled, flags = carry
        fdeg = flags[0]
        f1 = flags[1].reshape(1, 1)
        f2 = flags[2].reshape(1, 1)
        xin = jnp.where(fdeg == 1, ones2n, h.reshape(2 * N, 128))
        dinvin = jnp.where(fdeg == 1, ones_n, dinv.reshape(N))
        u = spmm(xin, src_s, dst_s, ew_s, dinvin)
        uq = (u[0, :N], u[1, :N])
        dinv_new, dinv2_new = _dinv_tc(u[0, :N])
        dinv = jnp.where(fdeg == 1, dinv_new, dinv)
        dinv2 = jnp.where(fdeg == 1, dinv2_new, dinv2)
        z = _mm1_tc(f1, uq, h, dinv2, W1, b1r, cw1p, cb1p, W2)
        pooled_new = _mm2_tc(f2, uq, h, dinv2, b2r, cw2p, cb2p, ibT)
        h = jnp.where(f1[0, 0] == 1, z, h)
        pooled = jnp.where(f2[0, 0] == 1, pooled_new, pooled)
        return (h, dinv, dinv2, pooled, jnp.roll(flags, 1))

    carry0 = (x, jnp.zeros((N, 1), _f32), jnp.zeros((N, 1), _f32),
              jnp.zeros((NSEG, D2), _f32),
              jnp.array([1, 0, 0], _i32))
    _, _, _, out, _ = lax.fori_loop(0, steps, _step, carry0)
    return out


# speedup vs baseline: 2.7167x; 1.0155x over previous
import functools

import jax
import jax.numpy as jnp
from jax import lax
from jax.experimental import pallas as pl
from jax.experimental.pallas import tpu as pltpu
from jax.experimental.pallas import tpu_sc as plsc

N = 10000
E = 160000
F = 256
D1 = 512
D2 = 256
NSEG = 64

NC = 2
NS = 16
LANES = 16

PASSES = 5
HALF = 2048
NPAD = PASSES * HALF
ACCR = HALF
EPAD = 163840
BATCH = 128
NB_S = EPAD // NS // BATCH
CH = 16

RB = 1000

_f32 = jnp.float32
_i32 = jnp.int32


@functools.lru_cache(maxsize=None)
def _sc_mesh():
    return plsc.VectorSubcoreMesh(
        core_axis_name="c", subcore_axis_name="s",
        num_cores=NC, num_subcores=NS)


def _spmm_body(x_hbm, src_hbm, dst_hbm, ew_hbm, dinv_hbm, out_hbm,
               sbuf, dbuf, ewb, gidx, didxp, gidxp, norms, normsp, gidxf,
               didxf, dinvL, rows, acc, semga):
    c = lax.axis_index("c")
    s = lax.axis_index("s")
    pltpu.sync_copy(src_hbm.at[s], sbuf)
    pltpu.sync_copy(dst_hbm.at[s], dbuf)
    pltpu.sync_copy(ew_hbm.at[s], ewb)
    pltpu.sync_copy(dinv_hbm, dinvL)

    def _pre(b, _):
        for k in range(BATCH // LANES):
            sl = pl.ds(k * LANES, LANES)
            sv = sbuf[b, sl]
            dv = dbuf[b, sl]
            ev = ewb[b, sl]
            nv = plsc.load_gather(dinvL, [sv]) * ev * plsc.load_gather(dinvL, [dv])
            norms[pl.ds(b * BATCH + k * LANES, LANES)] = nv
            gidx[b, sl] = sv * 2 + c
        return 0
    lax.fori_loop(0, NB_S, _pre, 0)


    def _pass(p, _):
        def _compact(g, cnt):
            b = g // (BATCH // LANES)
            k = g - b * (BATCH // LANES)
            sl = pl.ds(k * LANES, LANES)
            fl = pl.ds(g * LANES, LANES)
            lv = dbuf[b, sl] - p * HALF
            inh = (lv >= 0) & (lv < HALF)
            plsc.store_compressed(didxf.at[pl.ds(cnt, LANES)], lv, mask=inh)
            plsc.store_compressed(gidxf.at[pl.ds(cnt, LANES)],
                                  gidx[b, sl], mask=inh)
            plsc.store_compressed(normsp.at[pl.ds(cnt, LANES)],
                                  norms[fl], mask=inh)
            npop = plsc.all_reduce_population_count(inh)
            return cnt + jnp.max(npop)
        cnt = lax.fori_loop(0, NB_S * BATCH // LANES, _compact, 0)
        nb_p = (cnt + BATCH - 1) // BATCH

        def _tail(i, _):
            fl = pl.ds(cnt + i * LANES, LANES)
            gidxf[fl] = jnp.zeros((LANES,), _i32)
            normsp[fl] = jnp.zeros((LANES,), _f32)
            didxf[fl] = jnp.zeros((LANES,), _i32)
            return 0
        lax.fori_loop(0, BATCH // LANES, _tail, 0)

        def _zrow(i, _):
            for k in range(8):
                rows[i, pl.ds(k * LANES, LANES)] = jnp.zeros((LANES,), _f32)
            return 0
        lax.fori_loop(0, 32, _zrow, 0)
        zbase = s * (ACCR // NS)

        def _zcp(i, _):
            pltpu.sync_copy(rows.at[pl.ds(0, 32)],
                            acc.at[pl.ds(zbase + i * 32, 32)])
            return 0
        lax.fori_loop(0, ACCR // NS // 32, _zcp, 0)
        plsc.subcore_barrier()

        nq = (nb_p + CH - 1) // CH

        def _chunk(q, _):
            cbase = q * CH
            ncb = jnp.minimum(nb_p - cbase, CH)

            def _relay(bb, _):
                b = cbase + bb
                for k in range(BATCH // LANES):
                    sl = pl.ds(k * LANES, LANES)
                    fl = pl.ds(b * BATCH + k * LANES, LANES)
                    didxp[bb, sl] = didxf[fl]
                    gidxp[bb, sl] = gidxf[fl]
                return 0
            lax.fori_loop(0, ncb, _relay, 0)

            def _batch(bb, _):
                pltpu.async_copy(x_hbm.at[gidxp.at[bb]], rows, semga).wait()
                ebase = (cbase + bb) * BATCH

                def _edge8(jj, _):
                    j0 = jj * 8
                    sps = [plsc.load_gather(
                        normsp, [jnp.full((LANES,), ebase + j0 + e, _i32)])
                        for e in range(8)]
                    for e in range(8):
                        for k in range(8):
                            sl = pl.ds(k * LANES, LANES)
                            rows[j0 + e, sl] = rows[j0 + e, sl] * sps[e]
                    return 0
                lax.fori_loop(0, BATCH // 8, _edge8, 0)
                pltpu.sync_copy(rows, acc.at[didxp.at[bb]], add=True)
                return 0
            lax.fori_loop(0, ncb, _batch, 0)
            return 0
        lax.fori_loop(0, nq, _chunk, 0)
        plsc.subcore_barrier()

        obase = s * (HALF // NS)

        def _ocp(i, _):
            pltpu.sync_copy(
                acc.at[pl.ds(obase + i * 32, 32)],
                out_hbm.at[c, pl.ds(p * HALF + obase + i * 32, 32)])
            return 0
        lax.fori_loop(0, HALF // NS // 32, _ocp, 0)
        plsc.subcore_barrier()
        return 0

    lax.fori_loop(0, PASSES, _pass, 0)


@functools.lru_cache(maxsize=None)
def _spmm_kernel():
    return pl.kernel(
        _spmm_body,
        out_type=jax.ShapeDtypeStruct((NC, NPAD, 128), _f32),
        mesh=_sc_mesh(),
        compiler_params=pltpu.CompilerParams(needs_layout_passes=False),
        scratch_types=[
            pltpu.VMEM((NB_S, BATCH), _i32),
            pltpu.VMEM((NB_S, BATCH), _i32),
            pltpu.VMEM((NB_S, BATCH), _f32),
            pltpu.VMEM((NB_S, BATCH), _i32),
            pltpu.VMEM((CH, BATCH), _i32),
            pltpu.VMEM((CH, BATCH), _i32),
            pltpu.VMEM((NB_S * BATCH,), _f32),
            pltpu.VMEM((NB_S * BATCH + BATCH,), _f32),
            pltpu.VMEM((NB_S * BATCH + BATCH,), _i32),
            pltpu.VMEM((NB_S * BATCH + BATCH,), _i32),
            pltpu.VMEM((N,), _f32),
            pltpu.VMEM((BATCH, 128), _f32),
            pltpu.VMEM_SHARED((ACCR, 128), _f32),
            pltpu.SemaphoreType.DMA,
        ],
    )


def _dinv_body(d0_ref, dinv_ref, dinv2_ref):
    deg = 1.0 + d0_ref[...][:, 0:1]
    r = lax.rsqrt(deg)
    dinv_ref[...] = r
    dinv2_ref[...] = r * r


def _dinv_tc(d0):
    return pl.pallas_call(
        _dinv_body,
        out_shape=(jax.ShapeDtypeStruct((N, 1), _f32),
                   jax.ShapeDtypeStruct((N, 1), _f32)),
    )(d0)


def _mm1_body(flag_ref, ua_ref, ub_ref, x_ref, dinv2_ref, W1_ref, b1_ref,
              cw1_ref, cb1_ref, W2_ref, z_ref):
    @pl.when(flag_ref[0, 0] != 0)
    def _():
        y = jnp.concatenate([ua_ref[...], ub_ref[...]], axis=1)
        y = y + dinv2_ref[...] * x_ref[...]
        h = jnp.dot(y, W1_ref[...], preferred_element_type=_f32) + b1_ref[...]
        h = jnp.maximum(h, 0.0)
        z2 = jnp.zeros((RB, 2), _f32)
        hp = jnp.concatenate([z2, h, z2], axis=1)
        acc = jnp.broadcast_to(cb1_ref[:, 0:1], (RB, D1))
        for k in range(5):
            acc = acc + hp[:, k:k + D1] * cw1_ref[:, k:k + 1]
        h1 = jnp.maximum(acc, 0.0)
        z_ref[...] = jnp.dot(h1, W2_ref[...], preferred_element_type=_f32)


def _mm1_tc(flag, uq, x, dinv2, W1, b1r, cw1p, cb1p, W2):
    return pl.pallas_call(
        _mm1_body,
        grid=(N // RB,),
        in_specs=[
            pl.BlockSpec((1, 1), lambda i: (0, 0)),
            pl.BlockSpec((RB, 128), lambda i: (i, 0)),
            pl.BlockSpec((RB, 128), lambda i: (i, 0)),
            pl.BlockSpec((RB, F), lambda i: (i, 0)),
            pl.BlockSpec((RB, 1), lambda i: (i, 0)),
            pl.BlockSpec((F, D1), lambda i: (0, 0)),
            pl.BlockSpec((1, D1), lambda i: (0, 0)),
            pl.BlockSpec((1, 8), lambda i: (0, 0)),
            pl.BlockSpec((1, 8), lambda i: (0, 0)),
            pl.BlockSpec((D1, D2), lambda i: (0, 0)),
        ],
        out_specs=pl.BlockSpec((RB, D2), lambda i: (i, 0)),
        out_shape=jax.ShapeDtypeStruct((N, D2), _f32),
    )(flag, *uq, x, dinv2, W1, b1r, cw1p, cb1p, W2)


def _mm2_body(flag_ref, ua_ref, ub_ref, z_ref, dinv2_ref, b2_ref,
              cw2_ref, cb2_ref, ib_ref, out_ref, sums_acc, cnt_acc):
    i = pl.program_id(0)

    @pl.when(flag_ref[0, 0] != 0)
    def _():
        y = jnp.concatenate([ua_ref[...], ub_ref[...]], axis=1)
        y = y + dinv2_ref[...] * z_ref[...]
        o = jnp.maximum(y + b2_ref[...], 0.0)
        z1 = jnp.zeros((RB, 1), _f32)
        op = jnp.concatenate([z1, o, z1], axis=1)
        acc = jnp.broadcast_to(cb2_ref[:, 0:1], (RB, D2))
        for k in range(3):
            acc = acc + op[:, k:k + D2] * cw2_ref[:, k:k + 1]
        h2 = jnp.maximum(acc, 0.0)

        onehot = (lax.broadcasted_iota(_i32, (NSEG, RB), 0)
                  == ib_ref[0]).astype(_f32)

        @pl.when(i == 0)
        def _():
            sums_acc[...] = jnp.zeros_like(sums_acc)
            cnt_acc[...] = jnp.zeros_like(cnt_acc)

        sums_acc[...] += jnp.dot(onehot, h2, preferred_element_type=_f32)
        cnt_acc[...] += jnp.sum(onehot, axis=1, keepdims=True)

        @pl.when(i == (N // RB) - 1)
        def _():
            out_ref[...] = sums_acc[...] / jnp.maximum(cnt_acc[:, 0:1], 1.0)


def _mm2_tc(flag, uq, z, dinv2, b2r, cw2p, cb2p, ibT):
    return pl.pallas_call(
        _mm2_body,
        grid=(N // RB,),
        in_specs=[
            pl.BlockSpec((1, 1), lambda i: (0, 0)),
            pl.BlockSpec((RB, 128), lambda i: (i, 0)),
            pl.BlockSpec((RB, 128), lambda i: (i, 0)),
            pl.BlockSpec((RB, D2), lambda i: (i, 0)),
            pl.BlockSpec((RB, 1), lambda i: (i, 0)),
            pl.BlockSpec((1, D2), lambda i: (0, 0)),
            pl.BlockSpec((1, 8), lambda i: (0, 0)),
            pl.BlockSpec((1, 8), lambda i: (0, 0)),
            pl.BlockSpec((1, 1, RB), lambda i: (i, 0, 0)),
        ],
        out_specs=pl.BlockSpec((NSEG, D2), lambda i: (0, 0)),
        out_shape=jax.ShapeDtypeStruct((NSEG, D2), _f32),
        scratch_shapes=[pltpu.VMEM((NSEG, D2), _f32),
                        pltpu.VMEM((NSEG, 128), _f32)],
    )(flag, *uq, z, dinv2, b2r, cw2p, cb2p, ibT)


def kernel(x, edge_index, edge_attr, info_batch, W1, b1, cw1, cb1, W2, b2,
           cw2, cb2):
    src = edge_index[0].astype(_i32)
    dst = edge_index[1].astype(_i32)
    ew = edge_attr.astype(_f32)

    pad = EPAD - E
    srcp = jnp.concatenate([src, jnp.zeros((pad,), _i32)])
    dstp = jnp.concatenate([dst, jnp.zeros((pad,), _i32)])
    ewp = jnp.concatenate([ew, jnp.zeros((pad,), _f32)])

    src_s = srcp.reshape(NS, NB_S, BATCH)
    dst_s = dstp.reshape(NS, NB_S, BATCH)
    ew_s = ewp.reshape(NS, NB_S, BATCH)

    b1r = b1.reshape(1, D1)
    cw1p = jnp.concatenate([cw1, jnp.zeros((3,), _f32)]).reshape(1, 8)
    cb1p = jnp.broadcast_to(cb1.reshape(1, 1), (1, 8))
    b2r = b2.reshape(1, D2)
    cw2p = jnp.concatenate([cw2, jnp.zeros((5,), _f32)]).reshape(1, 8)
    cb2p = jnp.broadcast_to(cb2.reshape(1, 1), (1, 8))
    ibT = info_batch.astype(_i32).reshape(N // RB, 1, RB)

    spmm = _spmm_kernel()
    ones2n = jnp.ones((2 * N, 128), _f32)
    ones_n = jnp.ones((N,), _f32)

    steps = 2 + (ew[0] < 0.5).astype(_i32) + (ew[0] >= 0.5).astype(_i32)

    def _step(i, carry):
        h, dinv, dinv2, pooled, flags = carry
        fdeg = flags[0]
        f1 = flags[1].reshape(1, 1)
        f2 = flags[2].reshape(1, 1)
        xin = jnp.where(fdeg == 1, ones2n, h.reshape(2 * N, 128))
        dinvin = jnp.where(fdeg == 1, ones_n, dinv.reshape(N))
        u = spmm(xin, src_s, dst_s, ew_s, dinvin)
        uq = (u[0, :N], u[1, :N])
        dinv_new, dinv2_new = _dinv_tc(u[0, :N])
        dinv = jnp.where(fdeg == 1, dinv_new, dinv)
        dinv2 = jnp.where(fdeg == 1, dinv2_new, dinv2)
        z = _mm1_tc(f1, uq, h, dinv2, W1, b1r, cw1p, cb1p, W2)
        pooled_new = _mm2_tc(f2, uq, h, dinv2, b2r, cw2p, cb2p, ibT)
        h = jnp.where(f1[0, 0] == 1, z, h)
        pooled = jnp.where(f2[0, 0] == 1, pooled_new, pooled)
        return (h, dinv, dinv2, pooled, jnp.roll(flags, 1))

    carry0 = (x, jnp.zeros((N, 1), _f32), jnp.zeros((N, 1), _f32),
              jnp.zeros((NSEG, D2), _f32),
              jnp.array([1, 0, 0], _i32))
    _, _, _, out, _ = lax.fori_loop(0, steps, _step, carry0)
    return out
